# baseline (speedup 1.0000x reference)
import functools

import jax
import jax.numpy as jnp
from jax import lax
from jax.experimental import pallas as pl
from jax.experimental.pallas import tpu as pltpu
from jax.experimental.pallas import tpu_sc as plsc

_D = 32
_V = 257
_S = 2048
_MAXP = (_V - 1) // 2

_NC = 2
_NS = 16
_ROWS_PER_W = _S // (_NC * _NS)
_EPAD = 33
_STRIPE = 384
_NTILES = (_S - _STRIPE) // 128


def _band_body(emb_hbm, out_hbm, tmp, epad, bigbuf, sb0, sb1, sem_s, sem_c):
    c = lax.axis_index("c")
    s = lax.axis_index("s")
    base = (s * _NC + c) * _ROWS_PER_W

    pltpu.sync_copy(emb_hbm, tmp)
    lanes = lax.iota(jnp.int32, 16)

    def stage(pos, carry):
        epad[pl.ds(pos * _EPAD, 16)] = tmp[pl.ds(pos * _D, 16)]
        epad[pl.ds(pos * _EPAD + 16, 16)] = tmp[pl.ds(pos * _D + 16, 16)]
        return carry

    lax.fori_loop(0, _V, stage, 0)

    def gather_chunk(buf, i, j0, x0):
        pos = jnp.clip(j0 + lanes - i + _MAXP, 0, _V - 1) * _EPAD
        for d in range(_D):
            buf[d, pl.ds(x0, 16)] = plsc.load_gather(epad, [pos + d])

    def full_chunk(ci, carry):
        gather_chunk(bigbuf, base, ci * 16, ci * 16)
        return carry

    lax.fori_loop(0, _S // 16, full_chunk, 0)

    a_col = jnp.clip(((base - _MAXP) >> 7) << 7, 0, _S - _STRIPE)
    a_col = pl.multiple_of(a_col, 128)
    aidx = a_col >> 7

    def row(k, sb):
        i = base + k

        @pl.when(k >= 2)
        def _():
            pltpu.make_async_copy(
                sb, out_hbm.at[base, :, pl.ds(a_col, _STRIPE)], sem_s
            ).wait()

        def upd(u, carry):
            gather_chunk(sb, i, a_col + u * 16, u * 16)
            return carry

        lax.fori_loop(0, _STRIPE // 16, upd, 0)
        pltpu.async_copy(sb, out_hbm.at[i, :, pl.ds(a_col, _STRIPE)], sem_s)
        for t in range(_NTILES):
            @pl.when(t < aidx)
            def _():
                pltpu.async_copy(
                    bigbuf.at[:, pl.ds(128 * t, 128)],
                    out_hbm.at[i, :, pl.ds(128 * t, 128)],
                    sem_c,
                )

            @pl.when(t >= aidx)
            def _():
                pltpu.async_copy(
                    bigbuf.at[:, pl.ds(128 * t + _STRIPE, 128)],
                    out_hbm.at[i, :, pl.ds(128 * t + _STRIPE, 128)],
                    sem_c,
                )

        @pl.when(k >= 2)
        def _():
            for t in range(_NTILES):
                pltpu.make_async_copy(
                    bigbuf.at[:, pl.ds(0, 128)],
                    out_hbm.at[base, :, pl.ds(0, 128)],
                    sem_c,
                ).wait()

    def row_pair(k2, carry):
        row(2 * k2, sb0)
        row(2 * k2 + 1, sb1)
        return carry

    lax.fori_loop(0, _ROWS_PER_W // 2, row_pair, 0)

    for t in range(2 * _NTILES):
        pltpu.make_async_copy(
            bigbuf.at[:, pl.ds(0, 128)],
            out_hbm.at[base, :, pl.ds(0, 128)],
            sem_c,
        ).wait()
    for sb in (sb0, sb1):
        pltpu.make_async_copy(
            sb, out_hbm.at[base, :, pl.ds(a_col, _STRIPE)], sem_s
        ).wait()


@jax.jit
def _expand(emb_flat):
    mesh = plsc.VectorSubcoreMesh(core_axis_name="c", subcore_axis_name="s")
    call = functools.partial(
        pl.kernel,
        out_type=jax.ShapeDtypeStruct((_S, _D, _S), jnp.float32),
        mesh=mesh,
        compiler_params=pltpu.CompilerParams(needs_layout_passes=False),
        scratch_types=[
            pltpu.VMEM((_V * _D,), jnp.float32),
            pltpu.VMEM((_V * _EPAD,), jnp.float32),
            pltpu.VMEM((_D, _S), jnp.float32),
            pltpu.VMEM((_D, _STRIPE), jnp.float32),
            pltpu.VMEM((_D, _STRIPE), jnp.float32),
            pltpu.SemaphoreType.DMA,
            pltpu.SemaphoreType.DMA,
        ],
    )(_band_body)
    return call(emb_flat)


def kernel(q, v, embeddings):
    assert q.shape[1] == _S and v.shape[1] == _S
    assert embeddings.shape == (_V, _D)
    out_t = _expand(embeddings.reshape(-1))
    return jnp.swapaxes(out_t, 1, 2)
